# baseline (speedup 1.0000x reference)
import functools

import jax
import jax.numpy as jnp
from jax import lax
from jax.experimental import pallas as pl
from jax.experimental.pallas import tpu as pltpu
from jax.experimental.pallas import tpu_sc as plsc

_DIM = 64
_MARGIN = 0.5
_NC = 2
_NS = 16
_NW = _NC * _NS
_L = 16
_IDXCH = 128


def _sc_call(B):
    bpw = B // _NW
    nch = bpw // _IDXCH
    mesh = plsc.VectorSubcoreMesh(core_axis_name="c", subcore_axis_name="s")

    @functools.partial(
        pl.kernel,
        mesh=mesh,
        compiler_params=pltpu.CompilerParams(
            needs_layout_passes=False, use_tc_tiling_on_sc=False),
        out_type=[
            jax.ShapeDtypeStruct((_NW, _L), jnp.float32),
            jax.ShapeDtypeStruct((_NW, _L), jnp.float32),
        ],
        scratch_types=[
            pltpu.VMEM((nch, _IDXCH), jnp.int32),
            pltpu.VMEM((nch, _IDXCH), jnp.int32),
            pltpu.VMEM((nch, _IDXCH), jnp.int32),
            pltpu.VMEM((bpw, _DIM), jnp.float32),
            pltpu.VMEM((bpw, _DIM), jnp.float32),
            pltpu.VMEM((bpw, _DIM), jnp.float32),
            pltpu.VMEM((bpw,), jnp.float32),
            pltpu.VMEM((bpw,), jnp.float32),
            pltpu.VMEM((_L,), jnp.float32),
            pltpu.VMEM((_L,), jnp.float32),
            pltpu.SemaphoreType.DMA,
        ],
    )
    def sc_kernel(uid_hbm, pid_hbm, nid_hbm, ut_hbm, it_hbm, bt_hbm,
                  loss_out, l2_out,
                  uid_v, pid_v, nid_v, u_v, p_v, n_v, pb_v, nb_v,
                  loss_st, l2_st, sem):
        wid = lax.axis_index("s") * _NC + lax.axis_index("c")
        pltpu.sync_copy(uid_hbm.at[pl.ds(wid * nch, nch)], uid_v)
        pltpu.sync_copy(pid_hbm.at[pl.ds(wid * nch, nch)], pid_v)
        pltpu.sync_copy(nid_hbm.at[pl.ds(wid * nch, nch)], nid_v)
        copies = []
        for k in range(nch):
            sl = pl.ds(k * _IDXCH, _IDXCH)
            copies.append(pltpu.async_copy(ut_hbm.at[uid_v.at[k]], u_v.at[sl], sem))
            copies.append(pltpu.async_copy(it_hbm.at[pid_v.at[k]], p_v.at[sl], sem))
            copies.append(pltpu.async_copy(it_hbm.at[nid_v.at[k]], n_v.at[sl], sem))
            copies.append(pltpu.async_copy(bt_hbm.at[pid_v.at[k]], pb_v.at[sl], sem))
            copies.append(pltpu.async_copy(bt_hbm.at[nid_v.at[k]], nb_v.at[sl], sem))
        for c in copies:
            c.wait()

        def body(g, carry):
            loss_acc, sq_vec = carry
            pb = pb_v[pl.ds(g * _L, _L)]
            nb = nb_v[pl.ds(g * _L, _L)]
            for r in range(_L):
                b = g * _L + r
                app = jnp.zeros((_L,), jnp.float32)
                ann = jnp.zeros((_L,), jnp.float32)
                for c in range(_DIM // _L):
                    u = u_v[b, pl.ds(c * _L, _L)]
                    p = p_v[b, pl.ds(c * _L, _L)]
                    n = n_v[b, pl.ds(c * _L, _L)]
                    dp = u - p
                    dn = u - n
                    app = app + dp * dp
                    ann = ann + dn * dn
                    sq_vec = sq_vec + u * u + p * p + n * n
                rp = jnp.sum(app)
                rn = jnp.sum(ann)
                diff = (rn - rp) + (pb[r] - nb[r])
                hinge = jnp.maximum(_MARGIN - diff, jnp.float32(0.0))
                loss_acc = loss_acc + hinge
            return loss_acc, sq_vec

        loss_acc, sq_vec = lax.fori_loop(
            0, bpw // _L, body,
            (jnp.float32(0.0), jnp.zeros((_L,), jnp.float32)))
        lane0 = lax.iota(jnp.int32, _L) == 0
        loss_st[...] = jnp.where(lane0, loss_acc, jnp.float32(0.0))
        l2_st[...] = 0.5 * sq_vec
        pltpu.sync_copy(loss_st, loss_out.at[wid])
        pltpu.sync_copy(l2_st, l2_out.at[wid])

    return sc_kernel


def kernel(user_id, p_item_id, n_item_id, user_table, item_table, item_bias_table):
    B = user_id.shape[0]
    call = _sc_call(B)
    uid = user_id.astype(jnp.int32).reshape(-1, _IDXCH)
    pid = p_item_id.astype(jnp.int32).reshape(-1, _IDXCH)
    nid = n_item_id.astype(jnp.int32).reshape(-1, _IDXCH)
    bias = item_bias_table.reshape(-1)
    loss_p, l2_p = call(uid, pid, nid, user_table, item_table, bias)
    return jnp.sum(loss_p), jnp.sum(l2_p)
